# baseline (speedup 1.0000x reference)
import functools

import jax
import jax.numpy as jnp
from jax import lax
from jax.experimental import pallas as pl
from jax.experimental.pallas import tpu as pltpu
from jax.experimental.pallas import tpu_sc as plsc

N = 10000
E = 160000
D = 256
H = 128
NCORES = 2
NTILES = 16
CHUNK = 64
NP = 10240
EP = 163840
NCHUNK = EP // (NTILES * CHUNK)
NB = 4
G = 16
NGRP = NCHUNK // G
ROWS_PER_TILE = NP // NTILES
MB = 1024


def _mm1_body(x_ref, w_ref, o_ref):
    o_ref[0] = jnp.dot(x_ref[...], w_ref[...],
                       preferred_element_type=jnp.float32)


def _mm1(xp, w1):
    return pl.pallas_call(
        _mm1_body,
        grid=(NCORES, NP // MB),
        in_specs=[
            pl.BlockSpec((MB, D), lambda c, i: (i, 0)),
            pl.BlockSpec((D, H), lambda c, i: (0, c)),
        ],
        out_specs=pl.BlockSpec((1, MB, H), lambda c, i: (c, i, 0)),
        out_shape=jax.ShapeDtypeStruct((NCORES, NP, H), jnp.float32),
    )(xp, w1)


def _mm2_body(a_ref, w_ref, o_ref):
    h = jnp.concatenate([a_ref[0], a_ref[1]], axis=1)
    h = jnp.maximum(h, 0.0)
    r = jnp.dot(h, w_ref[...], preferred_element_type=jnp.float32)
    o_ref[0] = r[:, :H]
    o_ref[1] = r[:, H:]


def _mm2(a, w2):
    return pl.pallas_call(
        _mm2_body,
        grid=(NP // MB,),
        in_specs=[
            pl.BlockSpec((NCORES, MB, H), lambda i: (0, i, 0)),
            pl.BlockSpec((D, D), lambda i: (0, 0)),
        ],
        out_specs=pl.BlockSpec((NCORES, MB, H), lambda i: (0, i, 0)),
        out_shape=jax.ShapeDtypeStruct((NCORES, NP, H), jnp.float32),
    )(a, w2)


def _ls_body(a_ref, o_ref):
    h = jnp.concatenate([a_ref[0], a_ref[1]], axis=1)
    m = jnp.max(h, axis=1, keepdims=True)
    s = h - m
    lse = jnp.log(jnp.sum(jnp.exp(s), axis=1, keepdims=True))
    o_ref[...] = s - lse


def _log_softmax(a):
    return pl.pallas_call(
        _ls_body,
        grid=(NP // MB,),
        in_specs=[pl.BlockSpec((NCORES, MB, H), lambda i: (0, i, 0))],
        out_specs=pl.BlockSpec((MB, D), lambda i: (i, 0)),
        out_shape=jax.ShapeDtypeStruct((NP, D), jnp.float32),
    )(a)


def _spmm_tile(h_hbm, col_hbm, row_hbm, norm_hbm, out_hbm,
               colg, rowg, normg, r0, r1, r2, r3, acc,
               g0, g1, g2, g3, s0, s1, s2, s3):
    c = lax.axis_index("c")
    s = lax.axis_index("s")
    rows = (r0, r1, r2, r3)
    gsem = (g0, g1, g2, g3)
    ssem = (s0, s1, s2, s3)

    def zero_body(i, _):
        for j in range(H // 16):
            r0[i, pl.ds(16 * j, 16)] = jnp.zeros((16,), jnp.float32)
        return 0
    lax.fori_loop(0, CHUNK, zero_body, 0)
    for k in range(ROWS_PER_TILE // CHUNK):
        pltpu.sync_copy(
            r0, acc.at[pl.ds(s * ROWS_PER_TILE + k * CHUNK, CHUNK)])

    plsc.subcore_barrier()

    def _gather(kk, j):
        pltpu.async_copy(h_hbm.at[c].at[colg.at[kk]], rows[j], gsem[j])

    def _wait_gather(kk, j):
        pltpu.make_async_copy(h_hbm.at[c].at[colg.at[kk]], rows[j],
                              gsem[j]).wait()

    def _scatter(kk, j):
        pltpu.async_copy(rows[j], acc.at[rowg.at[kk]], ssem[j], add=True)

    def _wait_scatter(kk, j):
        pltpu.make_async_copy(rows[j], acc.at[rowg.at[kk]], ssem[j]).wait()

    def _scale(kk, j):
        def edge_body(e, _):
            nw = normg[pl.ds(kk * CHUNK + e, 16)]
            nv = jnp.full((16,), nw[0], jnp.float32)
            for q in range(H // 16):
                sl = pl.ds(16 * q, 16)
                rows[j][e, sl] = rows[j][e, sl] * nv
            return 0
        lax.fori_loop(0, CHUNK, edge_body, 0)

    def group_body(k, _):
        pltpu.sync_copy(col_hbm.at[s].at[k], colg)
        pltpu.sync_copy(row_hbm.at[s].at[k], rowg)
        pltpu.sync_copy(norm_hbm.at[s].at[k], normg.at[pl.ds(0, G * CHUNK)])

        for j in range(NB - 1):
            _gather(j, j)

        def quad_body(q, _):
            for j in range(NB):
                kk = NB * q + j
                _wait_gather(kk, j)
                _scale(kk, j)
                _scatter(kk, j)
                pj = (j - 1) % NB
                if j == 0:
                    @pl.when(q >= 1)
                    def _():
                        _wait_scatter(kk - 1, pj)
                        _gather(kk + NB - 1, pj)
                    @pl.when(q == 0)
                    def _():
                        _gather(kk + NB - 1, pj)
                else:
                    @pl.when(kk + NB - 1 <= G - 1)
                    def _():
                        _wait_scatter(kk - 1, pj)
                        _gather(kk + NB - 1, pj)
            return 0
        lax.fori_loop(0, G // NB, quad_body, 0)

        for j in range(NB):
            _wait_scatter(G - NB + j, j)
        return 0
    lax.fori_loop(0, NGRP, group_body, 0)

    plsc.subcore_barrier()

    pltpu.sync_copy(acc.at[pl.ds(s * ROWS_PER_TILE, ROWS_PER_TILE)],
                    out_hbm.at[c].at[pl.ds(s * ROWS_PER_TILE, ROWS_PER_TILE)])


def _spmm(h, colc, rowc, normc):
    mesh = plsc.VectorSubcoreMesh(core_axis_name="c", subcore_axis_name="s")
    kern = functools.partial(
        pl.kernel,
        out_type=jax.ShapeDtypeStruct((NCORES, NP, H), jnp.float32),
        mesh=mesh,
        scratch_types=[
            pltpu.VMEM((G, CHUNK), jnp.int32),
            pltpu.VMEM((G, CHUNK), jnp.int32),
            pltpu.VMEM((G * CHUNK + 16,), jnp.float32),
            pltpu.VMEM((CHUNK, H), jnp.float32),
            pltpu.VMEM((CHUNK, H), jnp.float32),
            pltpu.VMEM((CHUNK, H), jnp.float32),
            pltpu.VMEM((CHUNK, H), jnp.float32),
            pltpu.VMEM_SHARED((NP, H), jnp.float32),
            pltpu.SemaphoreType.DMA,
            pltpu.SemaphoreType.DMA,
            pltpu.SemaphoreType.DMA,
            pltpu.SemaphoreType.DMA,
            pltpu.SemaphoreType.DMA,
            pltpu.SemaphoreType.DMA,
            pltpu.SemaphoreType.DMA,
            pltpu.SemaphoreType.DMA,
        ],
    )(_spmm_tile)
    return kern(h, colc, rowc, normc)


def kernel(x, edge_index, norm, W1, W2):
    ei = edge_index.astype(jnp.int32)
    row = jnp.pad(ei[0], (0, EP - E)).reshape(NTILES, NGRP, G, CHUNK)
    col = jnp.pad(ei[1], (0, EP - E)).reshape(NTILES, NGRP, G, CHUNK)
    nrm = jnp.pad(norm, (0, EP - E)).reshape(NTILES, NGRP, G * CHUNK)
    xp = jnp.pad(x, ((0, NP - N), (0, 0)))

    h1 = _mm1(xp, W1)
    a1 = _spmm(h1, col, row, nrm)
    h2 = _mm2(a1, W2)
    a2 = _spmm(h2, col, row, nrm)
    out = _log_softmax(a2)
    return out[:N]


# speedup vs baseline: 1.0083x; 1.0083x over previous
import functools

import jax
import jax.numpy as jnp
from jax import lax
from jax.experimental import pallas as pl
from jax.experimental.pallas import tpu as pltpu
from jax.experimental.pallas import tpu_sc as plsc

N = 10000
E = 160000
D = 256
H = 128
NCORES = 2
NTILES = 16
CHUNK = 128
NP = 10240
EP = 163840
NCHUNK = EP // (NTILES * CHUNK)
G = 16
NGRP = NCHUNK // G
ROWS_PER_TILE = NP // NTILES
MB = 1024


def _mm1_body(x_ref, w_ref, o_ref):
    o_ref[0] = jnp.dot(x_ref[...], w_ref[...],
                       preferred_element_type=jnp.float32)


def _mm1(xp, w1):
    return pl.pallas_call(
        _mm1_body,
        grid=(NCORES, NP // MB),
        in_specs=[
            pl.BlockSpec((MB, D), lambda c, i: (i, 0)),
            pl.BlockSpec((D, H), lambda c, i: (0, c)),
        ],
        out_specs=pl.BlockSpec((1, MB, H), lambda c, i: (c, i, 0)),
        out_shape=jax.ShapeDtypeStruct((NCORES, NP, H), jnp.float32),
    )(xp, w1)


def _mm2_body(a_ref, w_ref, o_ref):
    h = jnp.concatenate([a_ref[0], a_ref[1]], axis=1)
    h = jnp.maximum(h, 0.0)
    r = jnp.dot(h, w_ref[...], preferred_element_type=jnp.float32)
    o_ref[0] = r[:, :H]
    o_ref[1] = r[:, H:]


def _mm2(a, w2):
    return pl.pallas_call(
        _mm2_body,
        grid=(NP // MB,),
        in_specs=[
            pl.BlockSpec((NCORES, MB, H), lambda i: (0, i, 0)),
            pl.BlockSpec((D, D), lambda i: (0, 0)),
        ],
        out_specs=pl.BlockSpec((NCORES, MB, H), lambda i: (0, i, 0)),
        out_shape=jax.ShapeDtypeStruct((NCORES, NP, H), jnp.float32),
    )(a, w2)


def _ls_body(a_ref, o_ref):
    h = jnp.concatenate([a_ref[0], a_ref[1]], axis=1)
    m = jnp.max(h, axis=1, keepdims=True)
    s = h - m
    lse = jnp.log(jnp.sum(jnp.exp(s), axis=1, keepdims=True))
    o_ref[...] = s - lse


def _log_softmax(a):
    return pl.pallas_call(
        _ls_body,
        grid=(NP // MB,),
        in_specs=[pl.BlockSpec((NCORES, MB, H), lambda i: (0, i, 0))],
        out_specs=pl.BlockSpec((MB, D), lambda i: (i, 0)),
        out_shape=jax.ShapeDtypeStruct((NP, D), jnp.float32),
    )(a)


def _spmm_tile(h_hbm, col_hbm, row_hbm, norm_hbm, out_hbm,
               colg, rowg, normg, r0, r1, acc,
               g0, g1, s0, s1):
    c = lax.axis_index("c")
    s = lax.axis_index("s")
    rows = (r0, r1)
    gsem = (g0, g1)
    ssem = (s0, s1)

    def zero_body(i, _):
        for j in range(H // 16):
            r0[i, pl.ds(16 * j, 16)] = jnp.zeros((16,), jnp.float32)
        return 0
    lax.fori_loop(0, CHUNK, zero_body, 0)
    for k in range(ROWS_PER_TILE // CHUNK):
        pltpu.sync_copy(
            r0, acc.at[pl.ds(s * ROWS_PER_TILE + k * CHUNK, CHUNK)])

    plsc.subcore_barrier()

    def _gather(kk, j):
        pltpu.async_copy(h_hbm.at[c].at[colg.at[kk]], rows[j], gsem[j])

    def _wait_gather(kk, j):
        pltpu.make_async_copy(h_hbm.at[c].at[colg.at[kk]], rows[j],
                              gsem[j]).wait()

    def _scatter(kk, j):
        pltpu.async_copy(rows[j], acc.at[rowg.at[kk]], ssem[j], add=True)

    def _wait_scatter(kk, j):
        pltpu.make_async_copy(rows[j], acc.at[rowg.at[kk]], ssem[j]).wait()

    def _scale(kk, j):
        def edge_body(e, _):
            nw = normg[pl.ds(kk * CHUNK + e, 16)]
            nv = jnp.full((16,), nw[0], jnp.float32)
            for q in range(H // 16):
                sl = pl.ds(16 * q, 16)
                rows[j][e, sl] = rows[j][e, sl] * nv
            return 0
        lax.fori_loop(0, CHUNK, edge_body, 0)

    def group_body(k, _):
        pltpu.sync_copy(col_hbm.at[s].at[k], colg)
        pltpu.sync_copy(row_hbm.at[s].at[k], rowg)
        pltpu.sync_copy(norm_hbm.at[s].at[k], normg.at[pl.ds(0, G * CHUNK)])

        _gather(0, 0)

        def pair_body(p, _):
            kk0 = 2 * p
            kk1 = 2 * p + 1

            _wait_gather(kk0, 0)
            @pl.when(p >= 1)
            def _():
                _wait_scatter(kk0 - 1, 1)
            _gather(kk1, 1)
            _scale(kk0, 0)
            _scatter(kk0, 0)

            _wait_gather(kk1, 1)
            _wait_scatter(kk0, 0)
            @pl.when(p <= G // 2 - 2)
            def _():
                _gather(kk1 + 1, 0)
            _scale(kk1, 1)
            _scatter(kk1, 1)
            return 0
        lax.fori_loop(0, G // 2, pair_body, 0)

        _wait_scatter(G - 1, 1)
        return 0
    lax.fori_loop(0, NGRP, group_body, 0)

    plsc.subcore_barrier()

    pltpu.sync_copy(acc.at[pl.ds(s * ROWS_PER_TILE, ROWS_PER_TILE)],
                    out_hbm.at[c].at[pl.ds(s * ROWS_PER_TILE, ROWS_PER_TILE)])


def _spmm(h, colc, rowc, normc):
    mesh = plsc.VectorSubcoreMesh(core_axis_name="c", subcore_axis_name="s")
    kern = functools.partial(
        pl.kernel,
        out_type=jax.ShapeDtypeStruct((NCORES, NP, H), jnp.float32),
        mesh=mesh,
        scratch_types=[
            pltpu.VMEM((G, CHUNK), jnp.int32),
            pltpu.VMEM((G, CHUNK), jnp.int32),
            pltpu.VMEM((G * CHUNK + 16,), jnp.float32),
            pltpu.VMEM((CHUNK, H), jnp.float32),
            pltpu.VMEM((CHUNK, H), jnp.float32),
            pltpu.VMEM_SHARED((NP, H), jnp.float32),
            pltpu.SemaphoreType.DMA,
            pltpu.SemaphoreType.DMA,
            pltpu.SemaphoreType.DMA,
            pltpu.SemaphoreType.DMA,
        ],
    )(_spmm_tile)
    return kern(h, colc, rowc, normc)


def kernel(x, edge_index, norm, W1, W2):
    ei = edge_index.astype(jnp.int32)
    row = jnp.pad(ei[0], (0, EP - E)).reshape(NTILES, NGRP, G, CHUNK)
    col = jnp.pad(ei[1], (0, EP - E)).reshape(NTILES, NGRP, G, CHUNK)
    nrm = jnp.pad(norm, (0, EP - E)).reshape(NTILES, NGRP, G * CHUNK)
    xp = jnp.pad(x, ((0, NP - N), (0, 0)))

    h1 = _mm1(xp, W1)
    a1 = _spmm(h1, col, row, nrm)
    h2 = _mm2(a1, W2)
    a2 = _spmm(h2, col, row, nrm)
    out = _log_softmax(a2)
    return out[:N]


# speedup vs baseline: 1.0170x; 1.0087x over previous
import functools

import jax
import jax.numpy as jnp
from jax import lax
from jax.experimental import pallas as pl
from jax.experimental.pallas import tpu as pltpu
from jax.experimental.pallas import tpu_sc as plsc

N = 10000
E = 160000
D = 256
H = 128
NCORES = 2
NTILES = 16
CHUNK = 128
NP = 10240
EP = 163840
NCHUNK = EP // (NTILES * CHUNK)
G = 16
NGRP = NCHUNK // G
ROWS_PER_TILE = NP // NTILES
MB = 1024


def _mm1_body(x_ref, w_ref, o_ref):
    o_ref[0] = jnp.dot(x_ref[...], w_ref[...],
                       preferred_element_type=jnp.float32)


def _mm1(xp, w1):
    return pl.pallas_call(
        _mm1_body,
        grid=(NCORES, NP // MB),
        in_specs=[
            pl.BlockSpec((MB, D), lambda c, i: (i, 0)),
            pl.BlockSpec((D, H), lambda c, i: (0, c)),
        ],
        out_specs=pl.BlockSpec((1, MB, H), lambda c, i: (c, i, 0)),
        out_shape=jax.ShapeDtypeStruct((NCORES, NP, H), jnp.float32),
    )(xp, w1)


def _mm2_body(a_ref, w_ref, o_ref):
    h = jnp.concatenate([a_ref[0], a_ref[1]], axis=1)
    h = jnp.maximum(h, 0.0)
    r = jnp.dot(h, w_ref[...], preferred_element_type=jnp.float32)
    o_ref[0] = r[:, :H]
    o_ref[1] = r[:, H:]


def _mm2(a, w2):
    return pl.pallas_call(
        _mm2_body,
        grid=(NP // MB,),
        in_specs=[
            pl.BlockSpec((NCORES, MB, H), lambda i: (0, i, 0)),
            pl.BlockSpec((D, D), lambda i: (0, 0)),
        ],
        out_specs=pl.BlockSpec((NCORES, MB, H), lambda i: (0, i, 0)),
        out_shape=jax.ShapeDtypeStruct((NCORES, NP, H), jnp.float32),
    )(a, w2)


def _ls_body(a_ref, o_ref):
    h = jnp.concatenate([a_ref[0], a_ref[1]], axis=1)
    m = jnp.max(h, axis=1, keepdims=True)
    s = h - m
    lse = jnp.log(jnp.sum(jnp.exp(s), axis=1, keepdims=True))
    o_ref[...] = s - lse


def _log_softmax(a):
    return pl.pallas_call(
        _ls_body,
        grid=(NP // MB,),
        in_specs=[pl.BlockSpec((NCORES, MB, H), lambda i: (0, i, 0))],
        out_specs=pl.BlockSpec((MB, D), lambda i: (i, 0)),
        out_shape=jax.ShapeDtypeStruct((NP, D), jnp.float32),
    )(a)


def _spmm_tile(h_hbm, col_hbm, row_hbm, norm_hbm, out_hbm,
               colg, rowg, normg, r0, r1, acc,
               g0, g1, s0, s1):
    c = lax.axis_index("c")
    s = lax.axis_index("s")
    rows = (r0, r1)
    gsem = (g0, g1)
    ssem = (s0, s1)

    def zero_body(i, _):
        for j in range(H // 16):
            r1[i, pl.ds(16 * j, 16)] = jnp.zeros((16,), jnp.float32)
        return 0
    lax.fori_loop(0, CHUNK, zero_body, 0)
    for k in range(ROWS_PER_TILE // CHUNK):
        pltpu.async_copy(
            r1, acc.at[pl.ds(s * ROWS_PER_TILE + k * CHUNK, CHUNK)], s0)

    def _gather(kk, j):
        pltpu.async_copy(h_hbm.at[c].at[colg.at[kk]], rows[j], gsem[j])

    def _wait_gather(kk, j):
        pltpu.make_async_copy(h_hbm.at[c].at[colg.at[kk]], rows[j],
                              gsem[j]).wait()

    def _scatter(kk, j):
        pltpu.async_copy(rows[j], acc.at[rowg.at[kk]], ssem[j], add=True)

    def _wait_scatter(kk, j):
        pltpu.make_async_copy(rows[j], acc.at[rowg.at[kk]], ssem[j]).wait()

    def _scale(kk, j):
        def edge_body(e, _):
            nw = normg[pl.ds(kk * CHUNK + e, 16)]
            nv = jnp.full((16,), nw[0], jnp.float32)
            for q in range(H // 16):
                sl = pl.ds(16 * q, 16)
                rows[j][e, sl] = rows[j][e, sl] * nv
            return 0
        lax.fori_loop(0, CHUNK, edge_body, 0)

    def _stage(k):
        pltpu.sync_copy(col_hbm.at[s].at[k], colg)
        pltpu.sync_copy(row_hbm.at[s].at[k], rowg)
        pltpu.sync_copy(norm_hbm.at[s].at[k], normg.at[pl.ds(0, G * CHUNK)])

    _stage(0)
    _gather(0, 0)
    for k in range(ROWS_PER_TILE // CHUNK):
        pltpu.make_async_copy(
            r1, acc.at[pl.ds(s * ROWS_PER_TILE + k * CHUNK, CHUNK)],
            s0).wait()

    plsc.subcore_barrier()

    def group_body(k, _):
        def pair_body(p, _):
            kk0 = 2 * p
            kk1 = 2 * p + 1

            _wait_gather(kk0, 0)
            @pl.when(p >= 1)
            def _():
                _wait_scatter(kk0 - 1, 1)
            _gather(kk1, 1)
            _scale(kk0, 0)
            _scatter(kk0, 0)

            _wait_gather(kk1, 1)
            _wait_scatter(kk0, 0)
            @pl.when(p <= G // 2 - 2)
            def _():
                _gather(kk1 + 1, 0)
            _scale(kk1, 1)
            _scatter(kk1, 1)
            return 0
        lax.fori_loop(0, G // 2, pair_body, 0)

        _wait_scatter(G - 1, 1)

        @pl.when(k <= NGRP - 2)
        def _():
            _stage(k + 1)
            _gather(0, 0)
        return 0
    lax.fori_loop(0, NGRP, group_body, 0)

    plsc.subcore_barrier()

    pltpu.sync_copy(acc.at[pl.ds(s * ROWS_PER_TILE, ROWS_PER_TILE)],
                    out_hbm.at[c].at[pl.ds(s * ROWS_PER_TILE, ROWS_PER_TILE)])


def _spmm(h, colc, rowc, normc):
    mesh = plsc.VectorSubcoreMesh(core_axis_name="c", subcore_axis_name="s")
    kern = functools.partial(
        pl.kernel,
        out_type=jax.ShapeDtypeStruct((NCORES, NP, H), jnp.float32),
        mesh=mesh,
        scratch_types=[
            pltpu.VMEM((G, CHUNK), jnp.int32),
            pltpu.VMEM((G, CHUNK), jnp.int32),
            pltpu.VMEM((G * CHUNK + 16,), jnp.float32),
            pltpu.VMEM((CHUNK, H), jnp.float32),
            pltpu.VMEM((CHUNK, H), jnp.float32),
            pltpu.VMEM_SHARED((NP, H), jnp.float32),
            pltpu.SemaphoreType.DMA,
            pltpu.SemaphoreType.DMA,
            pltpu.SemaphoreType.DMA,
            pltpu.SemaphoreType.DMA,
        ],
    )(_spmm_tile)
    return kern(h, colc, rowc, normc)


def kernel(x, edge_index, norm, W1, W2):
    ei = edge_index.astype(jnp.int32)
    row = jnp.pad(ei[0], (0, EP - E)).reshape(NTILES, NGRP, G, CHUNK)
    col = jnp.pad(ei[1], (0, EP - E)).reshape(NTILES, NGRP, G, CHUNK)
    nrm = jnp.pad(norm, (0, EP - E)).reshape(NTILES, NGRP, G * CHUNK)
    xp = jnp.pad(x, ((0, NP - N), (0, 0)))

    h1 = _mm1(xp, W1)
    a1 = _spmm(h1, col, row, nrm)
    h2 = _mm2(a1, W2)
    a2 = _spmm(h2, col, row, nrm)
    out = _log_softmax(a2)
    return out[:N]


# speedup vs baseline: 1.7946x; 1.7645x over previous
import functools

import jax
import jax.numpy as jnp
from jax import lax
from jax.experimental import pallas as pl
from jax.experimental.pallas import tpu as pltpu
from jax.experimental.pallas import tpu_sc as plsc

N = 10000
E = 160000
D = 256
H = 128
NCORES = 2
NTILES = 16
CHUNK = 125
NP = 10240
EP = 160000
NCHUNK = EP // (NTILES * CHUNK)
G = 16
NGRP = NCHUNK // G
ROWS_PER_TILE = NP // NTILES
MB = 1024


def _mm1_body(x_ref, w_ref, o_ref):
    o_ref[0] = jnp.dot(x_ref[...], w_ref[...],
                       preferred_element_type=jnp.float32)


def _mm1(xp, w1):
    return pl.pallas_call(
        _mm1_body,
        grid=(NCORES, NP // MB),
        in_specs=[
            pl.BlockSpec((MB, D), lambda c, i: (i, 0)),
            pl.BlockSpec((D, H), lambda c, i: (0, c)),
        ],
        out_specs=pl.BlockSpec((1, MB, H), lambda c, i: (c, i, 0)),
        out_shape=jax.ShapeDtypeStruct((NCORES, NP, H), jnp.float32),
    )(xp, w1)


def _mm2_body(a_ref, w_ref, o_ref):
    h = jnp.concatenate([a_ref[0], a_ref[1]], axis=1)
    h = jnp.maximum(h, 0.0)
    r = jnp.dot(h, w_ref[...], preferred_element_type=jnp.float32)
    o_ref[0] = r[:, :H]
    o_ref[1] = r[:, H:]


def _mm2(a, w2):
    return pl.pallas_call(
        _mm2_body,
        grid=(NP // MB,),
        in_specs=[
            pl.BlockSpec((NCORES, MB, H), lambda i: (0, i, 0)),
            pl.BlockSpec((D, D), lambda i: (0, 0)),
        ],
        out_specs=pl.BlockSpec((NCORES, MB, H), lambda i: (0, i, 0)),
        out_shape=jax.ShapeDtypeStruct((NCORES, NP, H), jnp.float32),
    )(a, w2)


def _ls_body(a_ref, o_ref):
    h = jnp.concatenate([a_ref[0], a_ref[1]], axis=1)
    m = jnp.max(h, axis=1, keepdims=True)
    s = h - m
    lse = jnp.log(jnp.sum(jnp.exp(s), axis=1, keepdims=True))
    o_ref[...] = s - lse


def _log_softmax(a):
    return pl.pallas_call(
        _ls_body,
        grid=(NP // MB,),
        in_specs=[pl.BlockSpec((NCORES, MB, H), lambda i: (0, i, 0))],
        out_specs=pl.BlockSpec((MB, D), lambda i: (i, 0)),
        out_shape=jax.ShapeDtypeStruct((NP, D), jnp.float32),
    )(a)


def _spmm_tile(h_hbm, col_hbm, row_hbm, norm_hbm, out_hbm,
               colg, rowg, normg, r0, r1, acc,
               g0, g1, s0, s1):
    c = lax.axis_index("c")
    s = lax.axis_index("s")
    rows = (r0, r1)
    gsem = (g0, g1)
    ssem = (s0, s1)

    def zero_body(i, _):
        for j in range(H // 16):
            r1[i, pl.ds(16 * j, 16)] = jnp.zeros((16,), jnp.float32)
        return 0
    lax.fori_loop(0, CHUNK, zero_body, 0)
    for k in range(ROWS_PER_TILE // CHUNK):
        pltpu.async_copy(
            r1, acc.at[pl.ds(s * ROWS_PER_TILE + k * CHUNK, CHUNK)], s0)
    REM = ROWS_PER_TILE - (ROWS_PER_TILE // CHUNK) * CHUNK
    if REM:
        pltpu.async_copy(
            r1.at[pl.ds(0, REM)],
            acc.at[pl.ds(s * ROWS_PER_TILE + ROWS_PER_TILE - REM, REM)], s0)

    def _gather(kk, j):
        pltpu.async_copy(h_hbm.at[c].at[colg.at[kk]], rows[j], gsem[j])

    def _wait_gather(kk, j):
        pltpu.make_async_copy(h_hbm.at[c].at[colg.at[kk]], rows[j],
                              gsem[j]).wait()

    def _scatter(kk, j):
        pltpu.async_copy(rows[j], acc.at[rowg.at[kk]], ssem[j], add=True)

    def _wait_scatter(kk, j):
        pltpu.make_async_copy(rows[j], acc.at[rowg.at[kk]], ssem[j]).wait()

    def _scale(kk, j):
        def edge_body(e, _):
            nw = normg[pl.ds(kk * CHUNK + e, 16)]
            nv = jnp.full((16,), nw[0], jnp.float32)
            for q in range(H // 16):
                sl = pl.ds(16 * q, 16)
                rows[j][e, sl] = rows[j][e, sl] * nv
            return 0
        lax.fori_loop(0, CHUNK, edge_body, 0)

    def _stage(k):
        pltpu.sync_copy(col_hbm.at[s].at[k], colg)
        pltpu.sync_copy(row_hbm.at[s].at[k], rowg)
        pltpu.sync_copy(norm_hbm.at[s].at[k], normg.at[pl.ds(0, 2048)])

    _stage(0)
    _gather(0, 0)
    for k in range(ROWS_PER_TILE // CHUNK):
        pltpu.make_async_copy(
            r1, acc.at[pl.ds(s * ROWS_PER_TILE + k * CHUNK, CHUNK)],
            s0).wait()
    if REM:
        pltpu.make_async_copy(
            r1.at[pl.ds(0, REM)],
            acc.at[pl.ds(s * ROWS_PER_TILE + ROWS_PER_TILE - REM, REM)],
            s0).wait()

    plsc.subcore_barrier()

    def group_body(k, _):
        def pair_body(p, _):
            kk0 = 2 * p
            kk1 = 2 * p + 1

            _wait_gather(kk0, 0)
            @pl.when(p >= 1)
            def _():
                _wait_scatter(kk0 - 1, 1)
            _gather(kk1, 1)
            _scale(kk0, 0)
            _scatter(kk0, 0)

            _wait_gather(kk1, 1)
            _wait_scatter(kk0, 0)
            @pl.when(p <= G // 2 - 2)
            def _():
                _gather(kk1 + 1, 0)
            _scale(kk1, 1)
            _scatter(kk1, 1)
            return 0
        lax.fori_loop(0, G // 2, pair_body, 0)

        _wait_scatter(G - 1, 1)

        @pl.when(k <= NGRP - 2)
        def _():
            _stage(k + 1)
            _gather(0, 0)
        return 0
    lax.fori_loop(0, NGRP, group_body, 0)

    plsc.subcore_barrier()

    pltpu.sync_copy(acc.at[pl.ds(s * ROWS_PER_TILE, ROWS_PER_TILE)],
                    out_hbm.at[c].at[pl.ds(s * ROWS_PER_TILE, ROWS_PER_TILE)])


def _spmm(h, colc, rowc, normc):
    mesh = plsc.VectorSubcoreMesh(core_axis_name="c", subcore_axis_name="s")
    kern = functools.partial(
        pl.kernel,
        out_type=jax.ShapeDtypeStruct((NCORES, NP, H), jnp.float32),
        mesh=mesh,
        scratch_types=[
            pltpu.VMEM((G, CHUNK), jnp.int32),
            pltpu.VMEM((G, CHUNK), jnp.int32),
            pltpu.VMEM((2048 + 16,), jnp.float32),
            pltpu.VMEM((CHUNK, H), jnp.float32),
            pltpu.VMEM((CHUNK, H), jnp.float32),
            pltpu.VMEM_SHARED((NP, H), jnp.float32),
            pltpu.SemaphoreType.DMA,
            pltpu.SemaphoreType.DMA,
            pltpu.SemaphoreType.DMA,
            pltpu.SemaphoreType.DMA,
        ],
    )(_spmm_tile)
    return kern(h, colc, rowc, normc)


def kernel(x, edge_index, norm, W1, W2):
    ei = edge_index.astype(jnp.int32)
    row = jnp.pad(ei[0], (0, EP - E)).reshape(NTILES, NGRP, G, CHUNK)
    col = jnp.pad(ei[1], (0, EP - E)).reshape(NTILES, NGRP, G, CHUNK)
    nrm = jnp.pad(
        norm.reshape(NTILES, NGRP, G * CHUNK), ((0, 0), (0, 0), (0, 48)))
    xp = jnp.pad(x, ((0, NP - N), (0, 0)))

    h1 = _mm1(xp, W1)
    a1 = _spmm(h1, col, row, nrm)
    h2 = _mm2(a1, W2)
    a2 = _spmm(h2, col, row, nrm)
    out = _log_softmax(a2)
    return out[:N]
